# Initial kernel scaffold; baseline (speedup 1.0000x reference)
import functools

import jax
import jax.numpy as jnp
from jax import lax
from jax.experimental import pallas as pl
from jax.experimental.pallas import tpu as pltpu
from jax.experimental.pallas import tpu_sc as plsc

D = 64
L = 16
NC = 2
NS = 16
NW = NC * NS
CHUNK = 512


def _rsqrt(x):
    bits = plsc.bitcast(x, jnp.int32)
    bits = jnp.int32(0x5F3759DF) - lax.shift_right_logical(bits, 1)
    y = plsc.bitcast(bits, jnp.float32)
    for _ in range(3):
        y = y * (1.5 - 0.5 * x * y * y)
    return y


def _make_call(n_ids, n_rows):
    per_w = n_ids // NW
    n_chunks = per_w // CHUNK
    mesh = plsc.VectorSubcoreMesh(core_axis_name="c", subcore_axis_name="s")

    @functools.partial(
        pl.kernel,
        mesh=mesh,
        out_type=jax.ShapeDtypeStruct((n_ids, D), jnp.float32),
        scratch_types=[
            pltpu.VMEM((CHUNK,), jnp.int32),
            pltpu.VMEM((CHUNK, D), jnp.float32),
            pltpu.VMEM((2 * D,), jnp.float32),
            pltpu.SemaphoreType.DMA,
        ],
    )
    def call(ids_hbm, table_hbm, gb_hbm, out_hbm, idx_v, rows_v, gb_v, sem):
        wid = lax.axis_index("s") * NC + lax.axis_index("c")
        base = wid * per_w
        pltpu.sync_copy(gb_hbm, gb_v)
        g_vecs = [gb_v[pl.ds(L * i, L)] for i in range(D // L)]
        b_vecs = [gb_v[pl.ds(D + L * i, L)] for i in range(D // L)]

        def chunk_body(g, carry):
            off = pl.multiple_of(base + g * CHUNK, 8)
            pltpu.sync_copy(ids_hbm.at[pl.ds(off, CHUNK)], idx_v)
            pltpu.async_copy(table_hbm.at[idx_v], rows_v, sem).wait()

            def row_body(r, c2):
                vs = [rows_v[r, pl.ds(L * i, L)] for i in range(D // L)]
                s = (vs[0] + vs[1]) + (vs[2] + vs[3])
                q = (vs[0] * vs[0] + vs[1] * vs[1]) + (
                    vs[2] * vs[2] + vs[3] * vs[3])
                mean = jnp.sum(s) * (1.0 / D)
                var = jnp.sum(q) * (1.0 / D) - mean * mean
                rstd = _rsqrt(jnp.full((L,), var + 1e-5, jnp.float32))
                mean_v = jnp.full((L,), mean, jnp.float32)
                for i in range(D // L):
                    y = (vs[i] - mean_v) * rstd * g_vecs[i] + b_vecs[i]
                    rows_v[r, pl.ds(L * i, L)] = y
                return c2

            lax.fori_loop(0, CHUNK, row_body, 0, unroll=False)
            pltpu.sync_copy(rows_v, out_hbm.at[pl.ds(off, CHUNK)])
            return carry

        lax.fori_loop(0, n_chunks, chunk_body, 0, unroll=False)

    return call


def kernel(stock_ids, table, gamma, beta):
    ids = stock_ids.reshape(-1).astype(jnp.int32)
    gb = jnp.concatenate([gamma, beta]).astype(jnp.float32)
    out = _make_call(ids.shape[0], table.shape[0])(ids, table, gb)
    return out.reshape(stock_ids.shape + (D,))


# speedup vs baseline: 1.1766x; 1.1766x over previous
import functools

import jax
import jax.numpy as jnp
from jax import lax
from jax.experimental import pallas as pl
from jax.experimental.pallas import tpu as pltpu
from jax.experimental.pallas import tpu_sc as plsc

D = 64
L = 16
NC = 2
NS = 16
NW = NC * NS
CHUNK = 512


def _shuf(x, idx):
    dn = lax.GatherDimensionNumbers(
        offset_dims=(), collapsed_slice_dims=(0,), start_index_map=(0,))
    return lax.gather(x, idx[:, None], dn, (1,),
                      mode=lax.GatherScatterMode.PROMISE_IN_BOUNDS)


def _allsum(x):
    idx = lax.iota(jnp.int32, L)
    for d in (8, 4, 2, 1):
        x = x + _shuf(x, jnp.bitwise_xor(idx, d))
    return x


def _rsqrt(x):
    bits = plsc.bitcast(x, jnp.int32)
    bits = jnp.int32(0x5F3759DF) - lax.shift_right_logical(bits, 1)
    y = plsc.bitcast(bits, jnp.float32)
    for _ in range(3):
        y = y * (1.5 - 0.5 * x * y * y)
    return y


def _make_call(n_ids, n_rows):
    per_w = n_ids // NW
    n_chunks = per_w // CHUNK
    mesh = plsc.VectorSubcoreMesh(core_axis_name="c", subcore_axis_name="s")

    @functools.partial(
        pl.kernel,
        mesh=mesh,
        out_type=jax.ShapeDtypeStruct((n_ids, D), jnp.float32),
        scratch_types=[
            pltpu.VMEM((CHUNK,), jnp.int32),
            pltpu.VMEM((CHUNK, D), jnp.float32),
            pltpu.VMEM((2 * D,), jnp.float32),
            pltpu.SemaphoreType.DMA,
        ],
        compiler_params=pltpu.CompilerParams(
            needs_layout_passes=False, use_tc_tiling_on_sc=False),
    )
    def call(ids_hbm, table_hbm, gb_hbm, out_hbm, idx_v, rows_v, gb_v, sem):
        wid = lax.axis_index("s") * NC + lax.axis_index("c")
        base = wid * per_w
        pltpu.sync_copy(gb_hbm, gb_v)
        g_vecs = [gb_v[pl.ds(L * i, L)] for i in range(D // L)]
        b_vecs = [gb_v[pl.ds(D + L * i, L)] for i in range(D // L)]

        def chunk_body(g, carry):
            off = pl.multiple_of(base + g * CHUNK, 8)
            pltpu.sync_copy(ids_hbm.at[pl.ds(off, CHUNK)], idx_v)
            pltpu.async_copy(table_hbm.at[idx_v], rows_v, sem).wait()

            def row_body(r, c2):
                vs = [rows_v[r, pl.ds(L * i, L)] for i in range(D // L)]
                s = (vs[0] + vs[1]) + (vs[2] + vs[3])
                q = (vs[0] * vs[0] + vs[1] * vs[1]) + (
                    vs[2] * vs[2] + vs[3] * vs[3])
                mean_v = _allsum(s) * (1.0 / D)
                var_v = _allsum(q) * (1.0 / D) - mean_v * mean_v
                rstd = _rsqrt(var_v + 1e-5)
                for i in range(D // L):
                    y = (vs[i] - mean_v) * rstd * g_vecs[i] + b_vecs[i]
                    rows_v[r, pl.ds(L * i, L)] = y
                return c2

            lax.fori_loop(0, CHUNK, row_body, 0, unroll=False)
            pltpu.sync_copy(rows_v, out_hbm.at[pl.ds(off, CHUNK)])
            return carry

        lax.fori_loop(0, n_chunks, chunk_body, 0, unroll=False)

    return call


def kernel(stock_ids, table, gamma, beta):
    ids = stock_ids.reshape(-1).astype(jnp.int32)
    gb = jnp.concatenate([gamma, beta]).astype(jnp.float32)
    out = _make_call(ids.shape[0], table.shape[0])(ids, table, gb)
    return out.reshape(stock_ids.shape + (D,))


# speedup vs baseline: 1.3868x; 1.1787x over previous
import functools

import jax
import jax.numpy as jnp
from jax import lax
from jax.experimental import pallas as pl
from jax.experimental.pallas import tpu as pltpu
from jax.experimental.pallas import tpu_sc as plsc

D = 64
L = 16
NC = 2
NS = 16
NW = NC * NS
CHUNK = 512


def _shuf(x, idx):
    dn = lax.GatherDimensionNumbers(
        offset_dims=(), collapsed_slice_dims=(0,), start_index_map=(0,))
    return lax.gather(x, idx[:, None], dn, (1,),
                      mode=lax.GatherScatterMode.PROMISE_IN_BOUNDS)


def _allsum(x):
    idx = lax.iota(jnp.int32, L)
    for d in (8, 4, 2, 1):
        x = x + _shuf(x, jnp.bitwise_xor(idx, d))
    return x


def _rsqrt(x):
    bits = plsc.bitcast(x, jnp.int32)
    bits = jnp.int32(0x5F3759DF) - lax.shift_right_logical(bits, 1)
    y = plsc.bitcast(bits, jnp.float32)
    for _ in range(3):
        y = y * (1.5 - 0.5 * x * y * y)
    return y


def _layernorm_chunk(rows_v, g_vecs, b_vecs):
    def row_body(r, c):
        vs = [rows_v[r, pl.ds(L * i, L)] for i in range(D // L)]
        s = (vs[0] + vs[1]) + (vs[2] + vs[3])
        q = (vs[0] * vs[0] + vs[1] * vs[1]) + (vs[2] * vs[2] + vs[3] * vs[3])
        mean_v = _allsum(s) * (1.0 / D)
        var_v = _allsum(q) * (1.0 / D) - mean_v * mean_v
        rstd = _rsqrt(var_v + 1e-5)
        for i in range(D // L):
            y = (vs[i] - mean_v) * rstd * g_vecs[i] + b_vecs[i]
            rows_v[r, pl.ds(L * i, L)] = y
        return c

    lax.fori_loop(0, CHUNK, row_body, 0, unroll=8)


def _make_call(n_ids, n_rows):
    per_w = n_ids // NW
    n_chunks = per_w // CHUNK
    assert n_chunks % 2 == 0
    n_pairs = n_chunks // 2
    mesh = plsc.VectorSubcoreMesh(core_axis_name="c", subcore_axis_name="s")

    @functools.partial(
        pl.kernel,
        mesh=mesh,
        out_type=jax.ShapeDtypeStruct((n_ids, D), jnp.float32),
        scratch_types=[
            pltpu.VMEM((CHUNK,), jnp.int32),
            pltpu.VMEM((CHUNK,), jnp.int32),
            pltpu.VMEM((CHUNK, D), jnp.float32),
            pltpu.VMEM((CHUNK, D), jnp.float32),
            pltpu.VMEM((2 * D,), jnp.float32),
            pltpu.SemaphoreType.DMA,
            pltpu.SemaphoreType.DMA,
            pltpu.SemaphoreType.DMA,
            pltpu.SemaphoreType.DMA,
        ],
        compiler_params=pltpu.CompilerParams(
            needs_layout_passes=False, use_tc_tiling_on_sc=False),
    )
    def call(ids_hbm, table_hbm, gb_hbm, out_hbm,
             idx0, idx1, rows0, rows1, gb_v, gsem0, gsem1, osem0, osem1):
        wid = lax.axis_index("s") * NC + lax.axis_index("c")
        base = wid * per_w
        pltpu.sync_copy(gb_hbm, gb_v)
        g_vecs = [gb_v[pl.ds(L * i, L)] for i in range(D // L)]
        b_vecs = [gb_v[pl.ds(D + L * i, L)] for i in range(D // L)]

        def off_of(g):
            return pl.multiple_of(base + g * CHUNK, 8)

        pltpu.sync_copy(ids_hbm.at[pl.ds(off_of(0), CHUNK)], idx0)
        pltpu.async_copy(table_hbm.at[idx0], rows0, gsem0)

        def pair_body(j, carry):
            g0 = 2 * j
            g1 = g0 + 1

            pltpu.sync_copy(ids_hbm.at[pl.ds(off_of(g1), CHUNK)], idx1)

            @pl.when(j > 0)
            def _():
                pltpu.make_async_copy(
                    rows1, out_hbm.at[pl.ds(off_of(g1 - 2), CHUNK)],
                    osem1).wait()

            pltpu.async_copy(table_hbm.at[idx1], rows1, gsem1)

            pltpu.make_async_copy(table_hbm.at[idx0], rows0, gsem0).wait()
            _layernorm_chunk(rows0, g_vecs, b_vecs)
            pltpu.async_copy(
                rows0, out_hbm.at[pl.ds(off_of(g0), CHUNK)], osem0)

            @pl.when(j < n_pairs - 1)
            def _():
                pltpu.sync_copy(ids_hbm.at[pl.ds(off_of(g0 + 2), CHUNK)],
                                idx0)
                pltpu.make_async_copy(
                    rows0, out_hbm.at[pl.ds(off_of(g0), CHUNK)],
                    osem0).wait()
                pltpu.async_copy(table_hbm.at[idx0], rows0, gsem0)

            pltpu.make_async_copy(table_hbm.at[idx1], rows1, gsem1).wait()
            _layernorm_chunk(rows1, g_vecs, b_vecs)
            pltpu.async_copy(
                rows1, out_hbm.at[pl.ds(off_of(g1), CHUNK)], osem1)
            return carry

        lax.fori_loop(0, n_pairs, pair_body, 0, unroll=False)

        pltpu.make_async_copy(
            rows0, out_hbm.at[pl.ds(off_of(n_chunks - 2), CHUNK)],
            osem0).wait()
        pltpu.make_async_copy(
            rows1, out_hbm.at[pl.ds(off_of(n_chunks - 1), CHUNK)],
            osem1).wait()

    return call


def kernel(stock_ids, table, gamma, beta):
    ids = stock_ids.reshape(-1).astype(jnp.int32)
    gb = jnp.concatenate([gamma, beta]).astype(jnp.float32)
    out = _make_call(ids.shape[0], table.shape[0])(ids, table, gb)
    return out.reshape(stock_ids.shape + (D,))


# speedup vs baseline: 1.4132x; 1.0190x over previous
import functools

import jax
import jax.numpy as jnp
from jax import lax
from jax.experimental import pallas as pl
from jax.experimental.pallas import tpu as pltpu
from jax.experimental.pallas import tpu_sc as plsc

D = 64
L = 16
NC = 2
NS = 16
NW = NC * NS
BCHUNK = 4


def _shuf(x, idx):
    dn = lax.GatherDimensionNumbers(
        offset_dims=(), collapsed_slice_dims=(0,), start_index_map=(0,))
    return lax.gather(x, idx[:, None], dn, (1,),
                      mode=lax.GatherScatterMode.PROMISE_IN_BOUNDS)


def _allsum(x):
    idx = lax.iota(jnp.int32, L)
    for d in (8, 4, 2, 1):
        x = x + _shuf(x, jnp.bitwise_xor(idx, d))
    return x


def _rsqrt(x):
    bits = plsc.bitcast(x, jnp.int32)
    bits = jnp.int32(0x5F3759DF) - lax.shift_right_logical(bits, 1)
    y = plsc.bitcast(bits, jnp.float32)
    for _ in range(2):
        y = y * (1.5 - 0.5 * x * y * y)
    return y


def _layernorm_chunk(rows_v, out_v, n, g_vecs, b_vecs):
    def row_body(r, c):
        vs = [rows_v[r, pl.ds(L * i, L)] for i in range(D // L)]
        t = (vs[0] + vs[1]) + (vs[2] + vs[3])
        q = (vs[0] * vs[0] + vs[1] * vs[1]) + (vs[2] * vs[2] + vs[3] * vs[3])
        mean_v = _allsum(t) * (1.0 / D)
        var_v = _allsum(q) * (1.0 / D) - mean_v * mean_v
        rstd = _rsqrt(var_v + 1e-5)
        for i in range(D // L):
            y = (vs[i] - mean_v) * rstd * g_vecs[i] + b_vecs[i]
            out_v[r, pl.ds(L * i, L)] = y
        return c

    lax.fori_loop(0, n, row_body, 0, unroll=8)


def _make_call(batch, seq, n_pairs_rows):
    bat_per_w = batch // NW
    n_chunks = bat_per_w // BCHUNK
    assert n_chunks % 2 == 0
    n_pairs = n_chunks // 2
    nlook = BCHUNK * seq
    mesh = plsc.VectorSubcoreMesh(core_axis_name="c", subcore_axis_name="s")

    @functools.partial(
        pl.kernel,
        mesh=mesh,
        out_type=jax.ShapeDtypeStruct((batch, seq, D), jnp.float32),
        scratch_types=[
            pltpu.VMEM((BCHUNK * seq,), jnp.int32),
            pltpu.VMEM((BCHUNK * seq,), jnp.int32),
            pltpu.VMEM((BCHUNK * seq, 2 * D), jnp.float32),
            pltpu.VMEM((BCHUNK * seq, 2 * D), jnp.float32),
            pltpu.VMEM((BCHUNK * seq, D), jnp.float32),
            pltpu.VMEM((BCHUNK * seq, D), jnp.float32),
            pltpu.VMEM((2 * D,), jnp.float32),
            pltpu.SemaphoreType.DMA,
            pltpu.SemaphoreType.DMA,
            pltpu.SemaphoreType.DMA,
            pltpu.SemaphoreType.DMA,
        ],
        compiler_params=pltpu.CompilerParams(
            needs_layout_passes=False, use_tc_tiling_on_sc=True),
    )
    def call(idp_hbm, table_hbm, gb_hbm, out_hbm,
             idx0, idx1, rows0, rows1, out0, out1, gb_v,
             gsem0, gsem1, osem0, osem1):
        wid = lax.axis_index("s") * NC + lax.axis_index("c")
        base = wid * bat_per_w
        pltpu.sync_copy(gb_hbm, gb_v)
        g_vecs = [gb_v[pl.ds(L * i, L)] for i in range(D // L)]
        b_vecs = [gb_v[pl.ds(D + L * i, L)] for i in range(D // L)]

        def brow_of(g):
            return pl.multiple_of(base + g * BCHUNK, 4)

        def copy_ids(g, idx_v):
            off = pl.multiple_of((base + g * BCHUNK) * seq, 8)
            pltpu.sync_copy(idp_hbm.at[pl.ds(off, nlook)], idx_v)

        def fire_out(g, out_v, osem):
            brow = brow_of(g)
            for b in range(BCHUNK):
                pltpu.async_copy(
                    out_v.at[pl.ds(b * seq, seq)], out_hbm.at[brow + b],
                    osem)

        def drain_out(out_v, osem):
            for b in range(BCHUNK):
                pltpu.make_async_copy(
                    out_hbm.at[0], out_v.at[pl.ds(b * seq, seq)], osem).wait()

        copy_ids(0, idx0)
        pltpu.async_copy(table_hbm.at[idx0], rows0, gsem0)

        def pair_body(j, carry):
            g0 = 2 * j
            g1 = g0 + 1

            copy_ids(g1, idx1)

            @pl.when(j > 0)
            def _():
                drain_out(out1, osem1)

            pltpu.async_copy(table_hbm.at[idx1], rows1, gsem1)

            pltpu.make_async_copy(table_hbm.at[idx0], rows0, gsem0).wait()
            _layernorm_chunk(rows0, out0, nlook, g_vecs, b_vecs)
            fire_out(g0, out0, osem0)

            @pl.when(j < n_pairs - 1)
            def _():
                copy_ids(g0 + 2, idx0)
                drain_out(out0, osem0)
                pltpu.async_copy(table_hbm.at[idx0], rows0, gsem0)

            pltpu.make_async_copy(table_hbm.at[idx1], rows1, gsem1).wait()
            _layernorm_chunk(rows1, out1, nlook, g_vecs, b_vecs)
            fire_out(g1, out1, osem1)
            return carry

        lax.fori_loop(0, n_pairs, pair_body, 0, unroll=False)

        drain_out(out0, osem0)
        drain_out(out1, osem1)

    return call


def kernel(stock_ids, table, gamma, beta):
    batch, seq = stock_ids.shape
    ids = stock_ids.reshape(-1).astype(jnp.int32)
    table2 = jnp.pad(table, ((0, 0), (0, D)))
    gb = jnp.concatenate([gamma, beta]).astype(jnp.float32)
    return _make_call(batch, seq, table2.shape[0])(ids, table2, gb)
